# baseline (speedup 1.0000x reference)
import functools

import jax
import jax.numpy as jnp
from jax import lax
from jax.experimental import pallas as pl
from jax.experimental.pallas import tpu as pltpu
from jax.experimental.pallas import tpu_sc as plsc

N = 10000
D = 128
NC = 2
NS = 16
NW = NC * NS
NPAD = 10240
RPT = NPAD // NS
CH = 128
BLK = 400

_mesh = plsc.VectorSubcoreMesh(core_axis_name="c", subcore_axis_name="s")


def _make_deg_kernel(nch):
    @functools.partial(
        pl.kernel,
        mesh=_mesh,
        out_type=jax.ShapeDtypeStruct((NC, NPAD), jnp.float32),
        scratch_types=[
            pltpu.VMEM((nch, CH), jnp.int32),
            pltpu.VMEM((CH,), jnp.float32),
            pltpu.VMEM((RPT,), jnp.float32),
            pltpu.VMEM_SHARED((NPAD,), jnp.float32),
            pltpu.SemaphoreType.DMA,
        ],
    )
    def deg_kernel(row_hbm, deg_hbm, idx_v, ones_v, zer_v, deg_sh, sem):
        c = lax.axis_index("c")
        s = lax.axis_index("s")
        wid = c * NS + s

        iload = pltpu.make_async_copy(
            row_hbm.at[pl.ds(wid * nch, nch)], idx_v, sem)
        iload.start()

        @pl.loop(0, CH, step=16)
        def _(i):
            ones_v[pl.ds(i, 16)] = jnp.full((16,), 1.0, dtype=jnp.float32)

        @pl.loop(0, RPT, step=16)
        def _(i):
            zer_v[pl.ds(i, 16)] = jnp.zeros((16,), dtype=jnp.float32)

        iload.wait()
        pltpu.sync_copy(zer_v, deg_sh.at[pl.ds(s * RPT, RPT)])
        plsc.subcore_barrier()

        @pl.loop(0, nch, step=16)
        def _(j):
            for t in range(16):
                pltpu.make_async_copy(
                    ones_v, deg_sh.at[idx_v.at[j + t]], sem).start(add=True)
            for t in range(16):
                pltpu.make_async_copy(
                    ones_v, deg_sh.at[idx_v.at[j + t]], sem).wait()

        plsc.subcore_barrier()
        pltpu.sync_copy(deg_sh.at[pl.ds(s * RPT, RPT)],
                        deg_hbm.at[c, pl.ds(s * RPT, RPT)])

    return deg_kernel


def _make_spmm_kernel(nch):
    nh = nch // 2

    @functools.partial(
        pl.kernel,
        mesh=_mesh,
        out_type=jax.ShapeDtypeStruct((NC, NPAD, D), jnp.float32),
        scratch_types=[
            pltpu.VMEM((nh, CH), jnp.int32),
            pltpu.VMEM((nh, CH), jnp.int32),
            pltpu.VMEM((CH, D), jnp.float32),
            pltpu.VMEM((CH, D), jnp.float32),
            pltpu.VMEM_SHARED((NPAD, D), jnp.float32),
            pltpu.SemaphoreType.DMA,
            pltpu.SemaphoreType.DMA,
            pltpu.SemaphoreType.DMA,
            pltpu.SemaphoreType.DMA,
        ],
    )
    def spmm_kernel(g_hbm, col_hbm, row_hbm, acc_hbm,
                    col_v, row_v, b0, b1, acc_sh,
                    gs0, gs1, ss0, ss1):
        c = lax.axis_index("c")
        s = lax.axis_index("s")
        wid = c * NS + s

        ic0 = pltpu.make_async_copy(
            col_hbm.at[pl.ds(wid * nch, nh)], col_v, ss0)
        ir0 = pltpu.make_async_copy(
            row_hbm.at[pl.ds(wid * nch, nh)], row_v, ss1)
        ic0.start()
        ir0.start()

        @pl.loop(0, 16)
        def _(r):
            @pl.loop(0, D, step=16)
            def _(j):
                b0[r, pl.ds(j, 16)] = jnp.zeros((16,), dtype=jnp.float32)

        zsrc = b0.at[pl.ds(0, 16)]

        @pl.loop(0, RPT, step=8 * 16)
        def _(r0):
            for t in range(8):
                pltpu.make_async_copy(
                    zsrc, acc_sh.at[pl.ds(s * RPT + r0 + t * 16, 16)],
                    gs0).start()
            for t in range(8):
                pltpu.make_async_copy(
                    zsrc, acc_sh.at[pl.ds(s * RPT + r0 + t * 16, 16)],
                    gs0).wait()

        plsc.subcore_barrier()

        def gather(j, buf, sem):
            return pltpu.make_async_copy(g_hbm.at[col_v.at[j]], buf, sem)

        def scatter(j, buf, sem):
            return pltpu.make_async_copy(buf, acc_sh.at[row_v.at[j]], sem)

        def process_half(h):
            base = wid * nch + h * nh
            if h == 0:
                ic0.wait()
                ir0.wait()
            else:
                pltpu.sync_copy(col_hbm.at[pl.ds(base, nh)], col_v)
                pltpu.sync_copy(row_hbm.at[pl.ds(base, nh)], row_v)

            gather(0, b0, gs0).start()
            gather(1, b1, gs1).start()

            @pl.loop(0, nh - 2, step=2)
            def _(j):
                gather(j, b0, gs0).wait()
                scatter(j, b0, ss0).start(add=True)
                gather(j + 1, b1, gs1).wait()
                scatter(j + 1, b1, ss1).start(add=True)
                scatter(j, b0, ss0).wait()
                gather(j + 2, b0, gs0).start()
                scatter(j + 1, b1, ss1).wait()
                gather(j + 3, b1, gs1).start()

            jl = nh - 2
            gather(jl, b0, gs0).wait()
            scatter(jl, b0, ss0).start(add=True)
            gather(jl + 1, b1, gs1).wait()
            scatter(jl + 1, b1, ss1).start(add=True)
            scatter(jl, b0, ss0).wait()
            scatter(jl + 1, b1, ss1).wait()

        process_half(0)
        process_half(1)

        plsc.subcore_barrier()
        pltpu.sync_copy(acc_sh.at[pl.ds(s * RPT, RPT)],
                        acc_hbm.at[c, pl.ds(s * RPT, RPT)])

    return spmm_kernel


def _tc_linear_body(x_ref, degs_ref, w_ref, b_ref, g_ref):
    h = lax.dot_general(
        x_ref[...], w_ref[...], (((1,), (1,)), ((), ())),
        preferred_element_type=jnp.float32) + b_ref[...]
    deg = degs_ref[0] + degs_ref[1] + 1.0
    dis = lax.rsqrt(deg)
    g_ref[...] = h * dis


def _tc_final_body(accs_ref, degs_ref, g_ref, out_ref):
    acc = accs_ref[0] + accs_ref[1]
    deg = degs_ref[0] + degs_ref[1] + 1.0
    dis = lax.rsqrt(deg)
    out_ref[...] = jnp.maximum((acc + g_ref[...]) * dis, 0.0)


def kernel(x, edge_index, W, b):
    e = edge_index.shape[1]
    nch = -(-(-(-e // CH) // NW) // 16) * 16
    e_pad = NW * nch * CH
    pad = e_pad - e

    row = edge_index[0]
    col = edge_index[1]
    if pad:
        pidx = jnp.arange(pad, dtype=jnp.int32)
        row = jnp.concatenate([row, N + (pidx % (NPAD - N))])
        col = jnp.concatenate([col, pidx % N])
    row2 = row.reshape(-1, CH)
    col2 = col.reshape(-1, CH)

    deg_parts = _make_deg_kernel(nch)(row2)
    deg3 = deg_parts[:, :, None]

    g = pl.pallas_call(
        _tc_linear_body,
        grid=(N // BLK,),
        in_specs=[
            pl.BlockSpec((BLK, D), lambda i: (i, 0)),
            pl.BlockSpec((NC, BLK, 1), lambda i: (0, i, 0)),
            pl.BlockSpec((D, D), lambda i: (0, 0)),
            pl.BlockSpec((1, D), lambda i: (0, 0)),
        ],
        out_specs=pl.BlockSpec((BLK, D), lambda i: (i, 0)),
        out_shape=jax.ShapeDtypeStruct((N, D), jnp.float32),
    )(x, deg3, W, b.reshape(1, D))

    accs = _make_spmm_kernel(nch)(g, col2, row2)

    out = pl.pallas_call(
        _tc_final_body,
        grid=(N // BLK,),
        in_specs=[
            pl.BlockSpec((NC, BLK, D), lambda i: (0, i, 0)),
            pl.BlockSpec((NC, BLK, 1), lambda i: (0, i, 0)),
            pl.BlockSpec((BLK, D), lambda i: (i, 0)),
        ],
        out_specs=pl.BlockSpec((BLK, D), lambda i: (i, 0)),
        out_shape=jax.ShapeDtypeStruct((N, D), jnp.float32),
    )(accs, deg3, g)

    return out


# speedup vs baseline: 1.1174x; 1.1174x over previous
import functools

import jax
import jax.numpy as jnp
import numpy as np
from jax import lax
from jax.experimental import pallas as pl
from jax.experimental.pallas import tpu as pltpu
from jax.experimental.pallas import tpu_sc as plsc

N = 10000
D = 128
NC = 2
NS = 16
NW = NC * NS
NPAD = 10240
RPT = NPAD // NS
CH = 128
BLK = 2000

_mesh = plsc.VectorSubcoreMesh(core_axis_name="c", subcore_axis_name="s")


def _make_deg_kernel(nch):
    @functools.partial(
        pl.kernel,
        mesh=_mesh,
        out_type=jax.ShapeDtypeStruct((NC, NPAD), jnp.float32),
        scratch_types=[
            pltpu.VMEM((nch, CH), jnp.int32),
            pltpu.VMEM((CH,), jnp.float32),
            pltpu.VMEM((RPT,), jnp.float32),
            pltpu.VMEM_SHARED((NPAD,), jnp.float32),
            pltpu.SemaphoreType.DMA,
        ],
    )
    def deg_kernel(row_hbm, deg_hbm, idx_v, ones_v, zer_v, deg_sh, sem):
        c = lax.axis_index("c")
        s = lax.axis_index("s")
        wid = c * NS + s

        iload = pltpu.make_async_copy(
            row_hbm.at[pl.ds(wid * nch, nch)], idx_v, sem)
        iload.start()

        @pl.loop(0, CH, step=16)
        def _(i):
            ones_v[pl.ds(i, 16)] = jnp.full((16,), 1.0, dtype=jnp.float32)

        @pl.loop(0, RPT, step=16)
        def _(i):
            zer_v[pl.ds(i, 16)] = jnp.zeros((16,), dtype=jnp.float32)

        iload.wait()
        pltpu.sync_copy(zer_v, deg_sh.at[pl.ds(s * RPT, RPT)])
        plsc.subcore_barrier()

        @pl.loop(0, nch, step=16)
        def _(j):
            for t in range(16):
                pltpu.make_async_copy(
                    ones_v, deg_sh.at[idx_v.at[j + t]], sem).start(add=True)
            for t in range(16):
                pltpu.make_async_copy(
                    ones_v, deg_sh.at[idx_v.at[j + t]], sem).wait()

        plsc.subcore_barrier()
        pltpu.sync_copy(deg_sh.at[pl.ds(s * RPT, RPT)],
                        deg_hbm.at[c, pl.ds(s * RPT, RPT)])

    return deg_kernel


def _make_spmm_kernel(nch):
    nh = nch // 2

    @functools.partial(
        pl.kernel,
        mesh=_mesh,
        out_type=jax.ShapeDtypeStruct((NC, NPAD, D), jnp.float32),
        scratch_types=[
            pltpu.VMEM((nh, CH), jnp.int32),
            pltpu.VMEM((nh, CH), jnp.int32),
            pltpu.VMEM((CH, D), jnp.float32),
            pltpu.VMEM((CH, D), jnp.float32),
            pltpu.VMEM_SHARED((NPAD, D), jnp.float32),
            pltpu.SemaphoreType.DMA,
            pltpu.SemaphoreType.DMA,
            pltpu.SemaphoreType.DMA,
            pltpu.SemaphoreType.DMA,
        ],
    )
    def spmm_kernel(g_hbm, col_hbm, row_hbm, acc_hbm,
                    col_v, row_v, b0, b1, acc_sh,
                    gs0, gs1, ss0, ss1):
        c = lax.axis_index("c")
        s = lax.axis_index("s")
        wid = c * NS + s

        ic0 = pltpu.make_async_copy(
            col_hbm.at[pl.ds(wid * nch, nh)], col_v, ss0)
        ir0 = pltpu.make_async_copy(
            row_hbm.at[pl.ds(wid * nch, nh)], row_v, ss1)
        ic0.start()
        ir0.start()

        @pl.loop(0, 16)
        def _(r):
            @pl.loop(0, D, step=16)
            def _(j):
                b0[r, pl.ds(j, 16)] = jnp.zeros((16,), dtype=jnp.float32)

        zsrc = b0.at[pl.ds(0, 16)]

        @pl.loop(0, RPT, step=8 * 16)
        def _(r0):
            for t in range(8):
                pltpu.make_async_copy(
                    zsrc, acc_sh.at[pl.ds(s * RPT + r0 + t * 16, 16)],
                    gs0).start()
            for t in range(8):
                pltpu.make_async_copy(
                    zsrc, acc_sh.at[pl.ds(s * RPT + r0 + t * 16, 16)],
                    gs0).wait()

        plsc.subcore_barrier()

        def gather(j, buf, sem):
            return pltpu.make_async_copy(g_hbm.at[col_v.at[j]], buf, sem)

        def scatter(j, buf, sem):
            return pltpu.make_async_copy(buf, acc_sh.at[row_v.at[j]], sem)

        def process_half(h):
            base = wid * nch + h * nh
            if h == 0:
                ic0.wait()
                ir0.wait()
            else:
                pltpu.sync_copy(col_hbm.at[pl.ds(base, nh)], col_v)
                pltpu.sync_copy(row_hbm.at[pl.ds(base, nh)], row_v)

            gather(0, b0, gs0).start()
            gather(1, b1, gs1).start()

            @pl.loop(0, nh - 2, step=2)
            def _(j):
                gather(j, b0, gs0).wait()
                scatter(j, b0, ss0).start(add=True)
                gather(j + 1, b1, gs1).wait()
                scatter(j + 1, b1, ss1).start(add=True)
                scatter(j, b0, ss0).wait()
                gather(j + 2, b0, gs0).start()
                scatter(j + 1, b1, ss1).wait()
                gather(j + 3, b1, gs1).start()

            jl = nh - 2
            gather(jl, b0, gs0).wait()
            scatter(jl, b0, ss0).start(add=True)
            gather(jl + 1, b1, gs1).wait()
            scatter(jl + 1, b1, ss1).start(add=True)
            scatter(jl, b0, ss0).wait()
            scatter(jl + 1, b1, ss1).wait()

        process_half(0)
        process_half(1)

        plsc.subcore_barrier()
        pltpu.sync_copy(acc_sh.at[pl.ds(s * RPT, RPT)],
                        acc_hbm.at[c, pl.ds(s * RPT, RPT)])

    return spmm_kernel


def _tc_linear_body(x_ref, degs_ref, w_ref, b_ref, g_ref):
    h = lax.dot_general(
        x_ref[...], w_ref[...], (((1,), (1,)), ((), ())),
        preferred_element_type=jnp.float32) + b_ref[...]
    degs = degs_ref[...]
    deg = degs[:, 0] + degs[:, 1] + 1.0
    dis = lax.rsqrt(deg)
    g_ref[...] = h * dis[:, None]


def _tc_final_body(accs_ref, degs_ref, g_ref, out_ref):
    acc = accs_ref[0] + accs_ref[1]
    degs = degs_ref[...]
    deg = degs[:, 0] + degs[:, 1] + 1.0
    dis = lax.rsqrt(deg)
    out_ref[...] = jnp.maximum((acc + g_ref[...]) * dis[:, None], 0.0)


def kernel(x, edge_index, W, b):
    e = edge_index.shape[1]
    nch = -(-(-(-e // CH) // NW) // 16) * 16
    e_pad = NW * nch * CH
    pad = e_pad - e

    row = edge_index[0]
    col = edge_index[1]
    if pad:
        pidx = np.arange(pad, dtype=np.int32)
        row = jnp.concatenate([row, jnp.asarray(N + (pidx % (NPAD - N)))])
        col = jnp.concatenate([col, jnp.asarray(pidx % N)])
    row2 = row.reshape(-1, CH)
    col2 = col.reshape(-1, CH)

    deg_parts = _make_deg_kernel(nch)(row2)
    deg_t = deg_parts.T

    g = pl.pallas_call(
        _tc_linear_body,
        grid=(N // BLK,),
        in_specs=[
            pl.BlockSpec((BLK, D), lambda i: (i, 0)),
            pl.BlockSpec((BLK, NC), lambda i: (i, 0)),
            pl.BlockSpec((D, D), lambda i: (0, 0)),
            pl.BlockSpec((1, D), lambda i: (0, 0)),
        ],
        out_specs=pl.BlockSpec((BLK, D), lambda i: (i, 0)),
        out_shape=jax.ShapeDtypeStruct((N, D), jnp.float32),
    )(x, deg_t, W, b.reshape(1, D))

    accs = _make_spmm_kernel(nch)(g, col2, row2)

    out = pl.pallas_call(
        _tc_final_body,
        grid=(N // BLK,),
        in_specs=[
            pl.BlockSpec((NC, BLK, D), lambda i: (0, i, 0)),
            pl.BlockSpec((BLK, NC), lambda i: (i, 0)),
            pl.BlockSpec((BLK, D), lambda i: (i, 0)),
        ],
        out_specs=pl.BlockSpec((BLK, D), lambda i: (i, 0)),
        out_shape=jax.ShapeDtypeStruct((N, D), jnp.float32),
    )(accs, deg_t, g)

    return out
